# baseline (speedup 1.0000x reference)
import functools

import jax
import jax.numpy as jnp
from jax import lax
from jax.experimental import pallas as pl
from jax.experimental.pallas import tpu as pltpu
from jax.experimental.pallas import tpu_sc as plsc

N_CONT = 13
N_CAT = 26
VOCAB = 100000
EDIM = 16
BATCH = 16384
EMB_W = N_CAT * EDIM
OUT_W = EMB_W + N_CONT

NC, NS, L = 2, 16, 16
NW = NC * NS
ROWS_PER_W = BATCH // NW
CHUNK = 128
N_CHUNKS = ROWS_PER_W // CHUNK
VECS_PER_COL = CHUNK // L
CONT_VECS = CHUNK * N_CONT // L


def _body(x_hbm, t_hbm, out_hbm, xin_v, idx_v, gbuf, cont_v, sem):
    wid = lax.axis_index("s") * NC + lax.axis_index("c")
    zeros = jnp.zeros((16,), jnp.int32)
    lanes = lax.iota(jnp.int32, 16)

    def chunk_body(ci, carry):
        base = wid * ROWS_PER_W + ci * CHUNK
        pltpu.sync_copy(x_hbm.at[pl.ds(base, CHUNK)], xin_v)

        def col_body(c, carry2):
            def vec_body(vm, carry3):
                m = vm * L + lanes
                cv = zeros + (c + N_CONT)
                val = plsc.load_gather(xin_v, [m, cv])
                idx_v[c, pl.ds(vm * L, L)] = val.astype(jnp.int32)
                return carry3

            lax.fori_loop(0, VECS_PER_COL, vec_body, 0)
            pltpu.async_copy(t_hbm.at[c].at[idx_v.at[c]], gbuf.at[c], sem)
            return carry2

        lax.fori_loop(0, N_CAT, col_body, 0)

        def cont_body(v, carry2):
            q = v * L + lanes
            r = q // N_CONT
            col = q - r * N_CONT
            val = plsc.load_gather(xin_v, [r, col])
            plsc.store_scatter(cont_v, [r, col], val)
            return carry2

        lax.fori_loop(0, CONT_VECS, cont_body, 0)

        def drain_body(c, carry2):
            pltpu.make_async_copy(
                t_hbm.at[c].at[idx_v.at[c]], gbuf.at[c], sem
            ).wait()
            return carry2

        lax.fori_loop(0, N_CAT, drain_body, 0)

        def write_body(c, carry2):
            pltpu.sync_copy(
                gbuf.at[c],
                out_hbm.at[pl.ds(base, CHUNK), pl.ds(c * EDIM, EDIM)],
            )
            return carry2

        lax.fori_loop(0, N_CAT, write_body, 0)
        pltpu.sync_copy(
            cont_v, out_hbm.at[pl.ds(base, CHUNK), pl.ds(EMB_W, L)]
        )
        return carry

    lax.fori_loop(0, N_CHUNKS, chunk_body, 0)


@jax.jit
def _cat_emb_head(x_in, tables):
    mesh = plsc.VectorSubcoreMesh(core_axis_name="c", subcore_axis_name="s")
    f = pl.kernel(
        _body,
        out_type=jax.ShapeDtypeStruct((BATCH, EMB_W + L), jnp.float32),
        mesh=mesh,
        scratch_types=[
            pltpu.VMEM((CHUNK, N_CONT + N_CAT), jnp.float32),
            pltpu.VMEM((N_CAT, CHUNK), jnp.int32),
            pltpu.VMEM((N_CAT, CHUNK, EDIM), jnp.float32),
            pltpu.VMEM((CHUNK, L), jnp.float32),
            pltpu.SemaphoreType.DMA,
        ],
        compiler_params=pltpu.CompilerParams(
            use_tc_tiling_on_sc=False, needs_layout_passes=False
        ),
    )
    out432 = f(x_in, tables)
    return out432[:, :OUT_W]


def kernel(x_in, tables):
    return _cat_emb_head(x_in, tables)


# speedup vs baseline: 1.4961x; 1.4961x over previous
import jax
import jax.numpy as jnp
from jax import lax
from jax.experimental import pallas as pl
from jax.experimental.pallas import tpu as pltpu
from jax.experimental.pallas import tpu_sc as plsc

N_CONT = 13
N_CAT = 26
VOCAB = 100000
EDIM = 16
BATCH = 16384
EMB_W = N_CAT * EDIM
OUT_W = EMB_W + N_CONT

NC, NS, L = 2, 16, 16
NW = NC * NS
ROWS_PER_W = BATCH // NW
CHUNK = 128
N_CHUNKS = ROWS_PER_W // CHUNK
TOTAL_CHUNKS = BATCH // CHUNK
CONT_VECS = CHUNK * N_CONT // L


def _body(t_hbm, idx_hbm, x_hbm, out_hbm, idx_v, dest_v, slab_v, xin_v, cont_v, sem):
    wid = lax.axis_index("s") * NC + lax.axis_index("c")
    lanes = lax.iota(jnp.int32, 16)

    def chunk_body(ci, carry):
        cid = wid * N_CHUNKS + ci
        base = cid * CHUNK
        pltpu.sync_copy(idx_hbm.at[cid], idx_v)
        pltpu.sync_copy(x_hbm.at[pl.ds(base, CHUNK)], xin_v)

        def fire_body(ce, carry2):
            pltpu.async_copy(t_hbm.at[idx_v.at[ce]], dest_v.at[ce], sem)
            return carry2

        lax.fori_loop(0, EMB_W, fire_body, 0)

        def drain_body(ce, carry2):
            pltpu.make_async_copy(
                t_hbm.at[idx_v.at[ce]], dest_v.at[ce], sem
            ).wait()
            return carry2

        lax.fori_loop(0, EMB_W, drain_body, 0)

        def slab_body(k, carry2):
            def tr_body(ce, carry3):
                val = dest_v[ce, pl.ds(k * L, L)]
                bvec = lanes
                cevec = lanes * 0 + ce
                plsc.store_scatter(slab_v, [bvec, cevec], val)
                return carry3

            lax.fori_loop(0, EMB_W, tr_body, 0)
            pltpu.sync_copy(
                slab_v, out_hbm.at[pl.ds(base + k * L, L), pl.ds(0, EMB_W)]
            )
            return carry2

        lax.fori_loop(0, CHUNK // L, slab_body, 0)

        def cont_body(v, carry2):
            q = v * L + lanes
            r = q // N_CONT
            col = q - r * N_CONT
            val = plsc.load_gather(xin_v, [r, col])
            plsc.store_scatter(cont_v, [r, col], val)
            return carry2

        lax.fori_loop(0, CONT_VECS, cont_body, 0)

        pltpu.sync_copy(cont_v, out_hbm.at[pl.ds(base, CHUNK), pl.ds(EMB_W, L)])
        return carry

    lax.fori_loop(0, N_CHUNKS, chunk_body, 0)


@jax.jit
def _cat_emb_head(x_in, tables):
    tswap = jnp.swapaxes(tables, 1, 2).reshape(EMB_W, VOCAB).reshape(-1)
    x_cat = x_in[:, N_CONT:].astype(jnp.int32)
    ce = jnp.arange(EMB_W, dtype=jnp.int32)
    v = x_cat[:, ce // EDIM]
    idx = ce[None, :] * VOCAB + v
    idx3 = idx.reshape(TOTAL_CHUNKS, CHUNK, EMB_W).transpose(0, 2, 1)

    mesh = plsc.VectorSubcoreMesh(core_axis_name="c", subcore_axis_name="s")
    f = pl.kernel(
        _body,
        out_type=jax.ShapeDtypeStruct((BATCH, EMB_W + L), jnp.float32),
        mesh=mesh,
        scratch_types=[
            pltpu.VMEM((EMB_W, CHUNK), jnp.int32),
            pltpu.VMEM((EMB_W, CHUNK), jnp.float32),
            pltpu.VMEM((L, EMB_W), jnp.float32),
            pltpu.VMEM((CHUNK, N_CONT + N_CAT), jnp.float32),
            pltpu.VMEM((CHUNK, L), jnp.float32),
            pltpu.SemaphoreType.DMA,
        ],
        compiler_params=pltpu.CompilerParams(
            use_tc_tiling_on_sc=False, needs_layout_passes=False
        ),
    )
    out432 = f(tswap, idx3, x_in)
    return out432[:, :OUT_W]


def kernel(x_in, tables):
    return _cat_emb_head(x_in, tables)


# speedup vs baseline: 1.6901x; 1.1297x over previous
import jax
import jax.numpy as jnp
from jax import lax
from jax.experimental import pallas as pl
from jax.experimental.pallas import tpu as pltpu
from jax.experimental.pallas import tpu_sc as plsc

N_CONT = 13
N_CAT = 26
VOCAB = 100000
EDIM = 16
BATCH = 16384
EMB_W = N_CAT * EDIM
OUT_W = EMB_W + N_CONT

NC, NS, L = 2, 16, 16
NW = NC * NS
ROWS_PER_W = BATCH // NW
CHUNK = 128
N_CHUNKS = ROWS_PER_W // CHUNK
TOTAL_CHUNKS = BATCH // CHUNK
CONT_VECS = CHUNK * N_CONT // L


def _body(t_hbm, idx_hbm, x_hbm, out_hbm, idx_v, dest_v, slab_v, xin_v, cont_v, sem):
    wid = lax.axis_index("s") * NC + lax.axis_index("c")
    lanes = lax.iota(jnp.int32, 16)

    def chunk_body(ci, carry):
        cid = wid * N_CHUNKS + ci
        base = cid * CHUNK
        pltpu.sync_copy(idx_hbm.at[cid], idx_v)
        pltpu.sync_copy(x_hbm.at[pl.ds(base, CHUNK)], xin_v)

        def fire_body(ce, carry2):
            pltpu.async_copy(t_hbm.at[idx_v.at[ce]], dest_v.at[ce], sem)
            return carry2

        lax.fori_loop(0, EMB_W, fire_body, 0)

        def cont_body(v, carry2):
            q = v * L + lanes
            r = q // N_CONT
            col = q - r * N_CONT
            val = plsc.load_gather(xin_v, [r, col])
            plsc.store_scatter(cont_v, [r, col], val)
            return carry2

        lax.fori_loop(0, CONT_VECS, cont_body, 0)
        pltpu.sync_copy(cont_v, out_hbm.at[pl.ds(base, CHUNK), pl.ds(EMB_W, L)])

        def half_body(h, carry2):
            ce0 = h * (EMB_W // 2)

            def drain_body(i, carry3):
                ce = ce0 + i
                pltpu.make_async_copy(
                    t_hbm.at[idx_v.at[ce]], dest_v.at[ce], sem
                ).wait()
                return carry3

            lax.fori_loop(0, EMB_W // 2, drain_body, 0)

            def slab_body(k, carry3):
                def tr_body(i, carry4):
                    ce = ce0 + i
                    val = dest_v[ce, pl.ds(k * L, L)]
                    cevec = lanes * 0 + i
                    plsc.store_scatter(slab_v, [lanes, cevec], val)
                    return carry4

                lax.fori_loop(0, EMB_W // 2, tr_body, 0)
                pltpu.sync_copy(
                    slab_v,
                    out_hbm.at[
                        pl.ds(base + k * L, L), pl.ds(ce0, EMB_W // 2)
                    ],
                )
                return carry3

            lax.fori_loop(0, CHUNK // L, slab_body, 0)
            return carry2

        lax.fori_loop(0, 2, half_body, 0)
        return carry

    lax.fori_loop(0, N_CHUNKS, chunk_body, 0)


@jax.jit
def _cat_emb_head(x_in, tables):
    tswap = jnp.swapaxes(tables, 1, 2).reshape(EMB_W, VOCAB).reshape(-1)
    x_cat = x_in[:, N_CONT:].astype(jnp.int32)
    ce = jnp.arange(EMB_W, dtype=jnp.int32)
    v = x_cat[:, ce // EDIM]
    idx = ce[None, :] * VOCAB + v
    idx3 = idx.reshape(TOTAL_CHUNKS, CHUNK, EMB_W).transpose(0, 2, 1)

    mesh = plsc.VectorSubcoreMesh(core_axis_name="c", subcore_axis_name="s")
    f = pl.kernel(
        _body,
        out_type=jax.ShapeDtypeStruct((BATCH, EMB_W + L), jnp.float32),
        mesh=mesh,
        scratch_types=[
            pltpu.VMEM((EMB_W, CHUNK), jnp.int32),
            pltpu.VMEM((EMB_W, CHUNK), jnp.float32),
            pltpu.VMEM((L, EMB_W // 2), jnp.float32),
            pltpu.VMEM((CHUNK, N_CONT + N_CAT), jnp.float32),
            pltpu.VMEM((CHUNK, L), jnp.float32),
            pltpu.SemaphoreType.DMA,
        ],
        compiler_params=pltpu.CompilerParams(
            use_tc_tiling_on_sc=False, needs_layout_passes=False
        ),
    )
    out432 = f(tswap, idx3, x_in)
    return out432[:, :OUT_W]


def kernel(x_in, tables):
    return _cat_emb_head(x_in, tables)
